# baseline (speedup 1.0000x reference)
import functools

import jax
import jax.numpy as jnp
from jax import lax
from jax.experimental import pallas as pl
from jax.experimental.pallas import tpu as pltpu
from jax.experimental.pallas import tpu_sc as plsc

B = 16384
D = 16
NC, NS = 2, 16
NW = NC * NS
RPW = B // NW
BLK = 128
NBLK = RPW // BLK
L = 16


def _mf_body(user_hbm, item_hbm, ue_hbm, ie_hbm, ub_hbm, ib_hbm, out_hbm,
             uidx_v, iidx_v, ue_v, ie_v, ub_v, ib_v, out_v, sem):
    wid = lax.axis_index("s") * NC + lax.axis_index("c")

    pltpu.sync_copy(user_hbm.at[pl.ds(wid * NBLK, NBLK)], uidx_v)
    pltpu.sync_copy(item_hbm.at[pl.ds(wid * NBLK, NBLK)], iidx_v)

    copies = []
    for j in range(NBLK):
        sl = pl.ds(j * BLK, BLK)
        copies.append(pltpu.async_copy(ue_hbm.at[uidx_v.at[j]], ue_v.at[sl], sem))
        copies.append(pltpu.async_copy(ie_hbm.at[iidx_v.at[j]], ie_v.at[sl], sem))
        copies.append(pltpu.async_copy(ub_hbm.at[uidx_v.at[j]], ub_v.at[sl], sem))
        copies.append(pltpu.async_copy(ib_hbm.at[iidx_v.at[j]], ib_v.at[sl], sem))
    for cp in copies:
        cp.wait()

    iota = lax.iota(jnp.int32, L)
    zeros = jnp.zeros((L,), jnp.int32)

    def chunk(t, _):
        base = t * L
        rows = base + iota
        acc = ub_v[pl.ds(base, L)] + ib_v[pl.ds(base, L)]
        parts = []
        for d in range(D):
            cols = jnp.full((L,), d, jnp.int32)
            u = plsc.load_gather(ue_v, [rows, cols])
            it = plsc.load_gather(ie_v, [rows, cols])
            parts.append(u * it)
        while len(parts) > 1:
            parts = [parts[k] + parts[k + 1] for k in range(0, len(parts), 2)]
        out_v[pl.ds(base, L)] = acc + parts[0]
        return _

    lax.fori_loop(0, RPW // L, chunk, None)

    pltpu.sync_copy(out_v, out_hbm.at[pl.ds(wid * RPW, RPW)])


@functools.partial(
    pl.kernel,
    out_type=jax.ShapeDtypeStruct((B,), jnp.float32),
    mesh=plsc.VectorSubcoreMesh(core_axis_name="c", subcore_axis_name="s"),
    compiler_params=pltpu.CompilerParams(
        needs_layout_passes=False, use_tc_tiling_on_sc=False),
    scratch_types=[
        pltpu.VMEM((NBLK, BLK), jnp.int32),
        pltpu.VMEM((NBLK, BLK), jnp.int32),
        pltpu.VMEM((RPW, D), jnp.float32),
        pltpu.VMEM((RPW, D), jnp.float32),
        pltpu.VMEM((RPW,), jnp.float32),
        pltpu.VMEM((RPW,), jnp.float32),
        pltpu.VMEM((RPW,), jnp.float32),
        pltpu.SemaphoreType.DMA,
    ],
)
def _mf_kernel(user2d, item2d, ue, ie, ub, ib, out,
               uidx_v, iidx_v, ue_v, ie_v, ub_v, ib_v, out_v, sem):
    _mf_body(user2d, item2d, ue, ie, ub, ib, out,
             uidx_v, iidx_v, ue_v, ie_v, ub_v, ib_v, out_v, sem)


def kernel(user, item, user_embeddings, item_embeddings, user_biases, item_biases):
    user2d = user.astype(jnp.int32).reshape(NW * NBLK, BLK)
    item2d = item.astype(jnp.int32).reshape(NW * NBLK, BLK)
    return _mf_kernel(user2d, item2d, user_embeddings, item_embeddings,
                      user_biases.reshape(-1), item_biases.reshape(-1))
